# baseline (speedup 1.0000x reference)
import functools

import jax
import jax.numpy as jnp
from jax import lax
from jax.experimental import pallas as pl
from jax.experimental.pallas import tpu as pltpu
from jax.experimental.pallas import tpu_sc as plsc

NN = 50000
EE = 800000
H = 64
HH = 32
NSUB = 16
KC = 128
SUP = 8
NSUP = 50
EPAD = NSUB * NSUP * SUP * KC
EPTP = EPAD // NSUB
NR3 = EPAD // (SUP * KC)
NACC = NN + 8
UR = 40
NUNITS = NN // UR

@functools.lru_cache(maxsize=None)
def _make_edge_seg_kernel():
    mesh = plsc.VectorSubcoreMesh(core_axis_name="c", subcore_axis_name="s")
    return functools.partial(
        pl.kernel,
        out_type=jax.ShapeDtypeStruct((2, NN, HH), jnp.float32),
        mesh=mesh,
        scratch_types=[
            pltpu.VMEM((SUP * KC,), jnp.int32),
            pltpu.VMEM((SUP * KC,), jnp.int32),
            pltpu.VMEM((SUP, KC), jnp.int32),
            pltpu.VMEM((SUP, KC), jnp.float32),
            pltpu.VMEM((16,), jnp.int32),
            pltpu.VMEM((KC, 16), jnp.int32),
            pltpu.VMEM((KC, 16), jnp.int32),
            pltpu.VMEM((KC, 16), jnp.int32),
            pltpu.VMEM((KC, 16), jnp.int32),
            pltpu.VMEM((KC, HH), jnp.float32),
            pltpu.VMEM((KC, HH), jnp.float32),
            pltpu.VMEM((UR, HH), jnp.float32),
            pltpu.VMEM_SHARED((NACC, HH), jnp.float32),
            pltpu.SemaphoreType.DMA,
            pltpu.SemaphoreType.DMA,
            pltpu.SemaphoreType.DMA,
            pltpu.SemaphoreType.DMA,
            pltpu.SemaphoreType.DMA,
            pltpu.SemaphoreType.DMA,
        ],
        compiler_params=pltpu.CompilerParams(needs_layout_passes=False, use_tc_tiling_on_sc=False),
    )(_edge_seg_body)


def _edge_seg_body(tD, tS, dstg, srcg, agg3, ea3, wa, out,
                   dbuf, sbuf, aggbuf, eabuf, wabuf, Ab0, Ab1, Bb0, Bb1,
                   Mb0, Mb1, Zb, acc, sga0, sga1, sgb0, sgb1, ssc0, ssc1):
    c = lax.axis_index("c")
    s = lax.axis_index("s")
    zero16 = jnp.zeros((16,), jnp.float32)
    Abs_ = (Ab0, Ab1)
    Bbs = (Bb0, Bb1)
    Mbs = (Mb0, Mb1)
    sgas = (sga0, sga1)
    sgbs = (sgb0, sgb1)
    sscs = (ssc0, ssc1)

    def zrow(i, carry):
        Zb[i, pl.ds(0, 16)] = zero16
        Zb[i, pl.ds(16, 16)] = zero16
        return carry
    lax.fori_loop(0, UR, zrow, 0)

    nunits = (NUNITS - s + NSUB - 1) // NSUB

    def zcp(i, carry):
        pltpu.sync_copy(Zb, acc.at[pl.ds((s + NSUB * i) * UR, UR)])
        return carry
    lax.fori_loop(0, nunits, zcp, 0)

    pltpu.sync_copy(wa.at[pl.ds(c * 16, 16)], wabuf)
    wab = plsc.bitcast(wabuf[pl.ds(0, 16)], jnp.bfloat16)
    plsc.subcore_barrier()

    def fire_gathers(k, b):
        sl = pl.ds(k * KC, KC)
        pltpu.async_copy(tD.at[dbuf.at[sl]], Abs_[b], sgas[b])
        pltpu.async_copy(tS.at[sbuf.at[sl]], Bbs[b], sgbs[b])

    def wait_gathers(k, b):
        sl = pl.ds(k * KC, KC)
        pltpu.make_async_copy(tD.at[dbuf.at[sl]], Abs_[b], sgas[b]).wait()
        pltpu.make_async_copy(tS.at[sbuf.at[sl]], Bbs[b], sgbs[b]).wait()

    def superbody(sp, carry):
        off = c * EPAD + s * EPTP + sp * (SUP * KC)
        r = s * NSUP + sp
        pltpu.sync_copy(dstg.at[pl.ds(off, SUP * KC)], dbuf)
        pltpu.sync_copy(srcg.at[pl.ds(off, SUP * KC)], sbuf)
        pltpu.sync_copy(agg3.at[r], aggbuf)
        pltpu.sync_copy(ea3.at[r], eabuf)
        fire_gathers(0, 0)

        def pairbody(p, carry2):
            for b in range(2):
                k = 2 * p + b
                kn = jnp.minimum(k + 1, SUP - 1)

                @pl.when(k < SUP - 1)
                def _():
                    fire_gathers(kn, 1 - b)

                wait_gathers(k, b)

                @pl.when(k >= 2)
                def _():
                    pltpu.make_async_copy(Mbs[b], acc.at[aggbuf.at[k]],
                                          sscs[b]).wait()

                Ab = Abs_[b]
                Bb = Bbs[b]
                Mb = Mbs[b]
                kvec = jnp.full((16,), k, jnp.int32)
                for e in range(KC):
                    eb = plsc.load_gather(
                        eabuf, [kvec, jnp.full((16,), e, jnp.int32)])
                    ebb = plsc.pack(eb, eb, format=plsc.PackFormat.INTERLEAVED)
                    a32 = plsc.bitcast(Ab[e, pl.ds(0, 16)], jnp.bfloat16)
                    b32 = plsc.bitcast(Bb[e, pl.ds(0, 16)], jnp.bfloat16)
                    m32 = jnp.maximum(a32 + b32 + ebb * wab, 0.0)
                    m0, m1 = plsc.unpack(m32,
                                         format=plsc.PackFormat.INTERLEAVED)
                    Mb[e, pl.ds(0, 16)] = m0
                    Mb[e, pl.ds(16, 16)] = m1
                pltpu.async_copy(Mb, acc.at[aggbuf.at[k]], sscs[b], add=True)
            return carry2
        lax.fori_loop(0, SUP // 2, pairbody, 0)

        for b in range(2):
            pltpu.make_async_copy(Mbs[b], acc.at[aggbuf.at[SUP - 2 + b]],
                                  sscs[b]).wait()
        return carry
    lax.fori_loop(0, NSUP, superbody, 0)

    plsc.subcore_barrier()

    def ocp(i, carry):
        rs = (s + NSUB * i) * UR
        pltpu.sync_copy(acc.at[pl.ds(rs, UR)], out.at[c, pl.ds(rs, UR)])
        return carry
    lax.fori_loop(0, nunits, ocp, 0)


_CW = 16


@functools.lru_cache(maxsize=None)
def _make_count_kernel():
    mesh = plsc.VectorSubcoreMesh(core_axis_name="c", subcore_axis_name="s")
    return functools.partial(
        pl.kernel,
        out_type=jax.ShapeDtypeStruct((2, NN, _CW), jnp.float32),
        mesh=mesh,
        scratch_types=[
            pltpu.VMEM((SUP, KC), jnp.int32),
            pltpu.VMEM((KC, _CW), jnp.float32),
            pltpu.VMEM((UR, _CW), jnp.float32),
            pltpu.VMEM_SHARED((NACC, _CW), jnp.float32),
        ],
        compiler_params=pltpu.CompilerParams(needs_layout_passes=False, use_tc_tiling_on_sc=False),
    )(_count_body)


def _count_body(agg4, out, aggbuf, Ob, Zb, acc):
    c = lax.axis_index("c")
    s = lax.axis_index("s")
    zero16 = jnp.zeros((16,), jnp.float32)
    one16 = jnp.ones((16,), jnp.float32)

    def fill(i, carry):
        Zb[i, pl.ds(0, 16)] = zero16
        return carry
    lax.fori_loop(0, UR, fill, 0)

    def fillo(i, carry):
        Ob[i, pl.ds(0, 16)] = one16
        return carry
    lax.fori_loop(0, KC, fillo, 0)

    nunits = (NUNITS - s + NSUB - 1) // NSUB

    def zcp(i, carry):
        pltpu.sync_copy(Zb, acc.at[pl.ds((s + NSUB * i) * UR, UR)])
        return carry
    lax.fori_loop(0, nunits, zcp, 0)
    plsc.subcore_barrier()

    def superbody(sp, carry):
        r = c * NR3 + s * NSUP + sp
        pltpu.sync_copy(agg4.at[r], aggbuf)

        def chunkbody(k, carry2):
            pltpu.sync_copy(Ob, acc.at[aggbuf.at[k]], add=True)
            return carry2
        lax.fori_loop(0, SUP, chunkbody, 0)
        return carry
    lax.fori_loop(0, NSUP, superbody, 0)

    plsc.subcore_barrier()

    def ocp(i, carry):
        rs = (s + NSUB * i) * UR
        pltpu.sync_copy(acc.at[pl.ds(rs, UR)], out.at[c, pl.ds(rs, UR)])
        return carry
    lax.fori_loop(0, nunits, ocp, 0)


_RB = 1000
_NG = NN // _RB


def _full(shape):
    return pl.BlockSpec(shape, lambda i: (0,) * len(shape))


def _rows(width):
    return pl.BlockSpec((_RB, width), lambda i: (i, 0))


def _project(x, W, b, extras):
    din = x.shape[1]
    ne = len(extras)

    def body(*refs):
        x_ref, W_ref, b_ref = refs[:3]
        prefs = refs[3:3 + 2 * ne]
        outs = refs[3 + 2 * ne:]
        y = jnp.dot(x_ref[...], W_ref[...],
                    preferred_element_type=jnp.float32) + b_ref[...]
        outs[0][...] = y
        for t in range(ne):
            outs[1 + t][...] = jnp.dot(
                y, prefs[2 * t][...],
                preferred_element_type=jnp.float32) + prefs[2 * t + 1][...]

    in_specs = [_rows(din), _full((din, H)), _full((1, H))]
    args = [x, W, b.reshape(1, H)]
    for (P, p) in extras:
        in_specs += [_full((H, H)), _full((1, H))]
        args += [P, p.reshape(1, H)]
    out_shapes = tuple(jax.ShapeDtypeStruct((NN, H), jnp.float32)
                       for _ in range(1 + ne))
    out_specs = tuple(_rows(H) for _ in range(1 + ne))
    return pl.pallas_call(
        body, grid=(_NG,), in_specs=in_specs, out_specs=out_specs,
        out_shape=out_shapes)(*args)


def _node_update(x, seg, cnt, W2, b2, M1, bm1, g, bln, M2, bm2, extras):
    ne = len(extras)

    def body(*refs):
        (x_ref, s0_ref, s1_ref, cnt_ref, W2_ref, b2_ref, M1a_ref, M1b_ref,
         bm1_ref, g_ref, bln_ref, M2_ref, bm2_ref) = refs[:13]
        prefs = refs[13:13 + 2 * ne]
        outs = refs[13 + 2 * ne:]
        cntv = cnt_ref[...]
        seg_ = jnp.concatenate([s0_ref[...], s1_ref[...]], axis=1)
        msg = (jnp.dot(seg_ / jnp.maximum(cntv, 1.0), W2_ref[...],
                       preferred_element_type=jnp.float32)
               + b2_ref[...] * (cntv > 0))
        t = (jnp.dot(x_ref[...], M1a_ref[...],
                     preferred_element_type=jnp.float32)
             + jnp.dot(msg, M1b_ref[...], preferred_element_type=jnp.float32)
             + bm1_ref[...])
        mu = jnp.mean(t, axis=-1, keepdims=True)
        var = jnp.mean((t - mu) ** 2, axis=-1, keepdims=True)
        h = jnp.maximum(
            g_ref[...] * (t - mu) / jnp.sqrt(var + 1e-5) + bln_ref[...], 0.0)
        y = jnp.dot(h, M2_ref[...],
                    preferred_element_type=jnp.float32) + bm2_ref[...]
        outs[0][...] = y
        for k in range(ne):
            outs[1 + k][...] = jnp.dot(
                y, prefs[2 * k][...],
                preferred_element_type=jnp.float32) + prefs[2 * k + 1][...]

    in_specs = [_rows(H), _rows(HH), _rows(HH), pl.BlockSpec((_RB, 1), lambda i: (i, 0)),
                _full((H, H)), _full((1, H)), _full((H, H)), _full((H, H)),
                _full((1, H)), _full((1, H)), _full((1, H)), _full((H, H)),
                _full((1, H))]
    args = [x, seg[0], seg[1], cnt, W2, b2.reshape(1, H), M1[:H], M1[H:],
            bm1.reshape(1, H), g.reshape(1, H), bln.reshape(1, H), M2,
            bm2.reshape(1, H)]
    for (P, p) in extras:
        in_specs += [_full((H, H)), _full((1, H))]
        args += [P, p.reshape(1, H)]
    out_shapes = tuple(jax.ShapeDtypeStruct((NN, H), jnp.float32)
                       for _ in range(1 + ne))
    out_specs = tuple(_rows(H) for _ in range(1 + ne))
    return pl.pallas_call(
        body, grid=(_NG,), in_specs=in_specs, out_specs=out_specs,
        out_shape=out_shapes)(*args)


def _pack_half(t32):
    lo = lax.bitcast_convert_type(t32[:, :16].astype(jnp.bfloat16),
                                  jnp.uint16).astype(jnp.uint32)
    hi = lax.bitcast_convert_type(t32[:, 16:].astype(jnp.bfloat16),
                                  jnp.uint16).astype(jnp.uint32)
    return (lo | (hi << 16)).astype(jnp.int32)


def _pack_table(t):
    return jnp.concatenate([_pack_half(t[:, :HH]), _pack_half(t[:, HH:])],
                           axis=0)


def _edge_phase(tableD, tableS, wa, dstg, srcg, agg3, ea3):
    wa_p = jnp.concatenate([_pack_half(wa[None, :HH]),
                            _pack_half(wa[None, HH:])], axis=0).reshape(-1)
    seg2 = _make_edge_seg_kernel()(_pack_table(tableD), _pack_table(tableS),
                                   dstg, srcg, agg3, ea3, wa_p)
    return seg2


def kernel(constraint_features, variable_features, edge_index, edge_attr,
           params):
    src = edge_index[0].astype(jnp.int32)
    dst = edge_index[1].astype(jnp.int32)
    ea = edge_attr.astype(jnp.float32)

    npad = EPAD - EE
    zpad = jnp.zeros((npad,), jnp.int32)
    dstp = jnp.concatenate([dst, zpad])
    srcp = jnp.concatenate([src, zpad])
    dsta = jnp.concatenate([dst, jnp.full((npad,), NN, jnp.int32)])
    srca = jnp.concatenate([src, jnp.full((npad,), NN, jnp.int32)])
    eap = jnp.concatenate([ea, jnp.zeros((npad,), jnp.float32)])
    dstg = jnp.concatenate([dstp, dstp + NN])
    srcg = jnp.concatenate([srcp, srcp + NN])
    dst3 = dsta.reshape(NR3, SUP, KC)
    src3 = srca.reshape(NR3, SUP, KC)
    ea3 = eap.reshape(NR3, SUP, KC)

    cnts = _make_count_kernel()(
        jnp.concatenate([dst3, src3]))
    cnt_c = cnts[0, :, 0:1]
    cnt_v = cnts[1, :, 0:1]

    lp0, lp1 = params['layers']
    W1_0 = lp0['cmsg1']['W']
    V1_0 = lp0['vmsg1']['W']
    W1_1 = lp1['cmsg1']['W']
    V1_1 = lp1['vmsg1']['W']

    cf = jnp.pad(constraint_features, ((0, 0), (0, 3)))
    vf = jnp.pad(variable_features, ((0, 0), (0, 5)))
    Wc = jnp.pad(params['cin']['W'], ((0, 3), (0, 0)))
    Wv = jnp.pad(params['vin']['W'], ((0, 5), (0, 0)))
    ch, A1 = _project(cf, Wc, params['cin']['b'],
                      [(W1_0[:H], jnp.zeros((H,), jnp.float32))])
    vh, B1, B2 = _project(
        vf, Wv, params['vin']['b'],
        [(W1_0[H:2 * H], lp0['cmsg1']['b']),
         (V1_0[:H], lp0['vmsg1']['b'])])

    seg = _edge_phase(A1, B1, W1_0[2 * H], dstg, srcg, dst3, ea3)
    ch, A2, A1n = _node_update(
        ch, seg, cnt_c, lp0['cmsg2']['W'], lp0['cmsg2']['b'],
        lp0['cmlp1']['W'], lp0['cmlp1']['b'], lp0['cln_g'], lp0['cln_b'],
        lp0['cmlp2']['W'], lp0['cmlp2']['b'],
        [(V1_0[H:2 * H], jnp.zeros((H,), jnp.float32)),
         (W1_1[:H], jnp.zeros((H,), jnp.float32))])

    seg = _edge_phase(A2, B2, V1_0[2 * H], dstg, srcg, src3, ea3)
    vh, B1n, B2n = _node_update(
        vh, seg, cnt_v, lp0['vmsg2']['W'], lp0['vmsg2']['b'],
        lp0['vmlp1']['W'], lp0['vmlp1']['b'], lp0['vln_g'], lp0['vln_b'],
        lp0['vmlp2']['W'], lp0['vmlp2']['b'],
        [(W1_1[H:2 * H], lp1['cmsg1']['b']),
         (V1_1[:H], lp1['vmsg1']['b'])])

    seg = _edge_phase(A1n, B1n, W1_1[2 * H], dstg, srcg, dst3, ea3)
    ch, A2n, out_c = _node_update(
        ch, seg, cnt_c, lp1['cmsg2']['W'], lp1['cmsg2']['b'],
        lp1['cmlp1']['W'], lp1['cmlp1']['b'], lp1['cln_g'], lp1['cln_b'],
        lp1['cmlp2']['W'], lp1['cmlp2']['b'],
        [(V1_1[H:2 * H], jnp.zeros((H,), jnp.float32)),
         (params['cout']['W'], params['cout']['b'])])

    seg = _edge_phase(A2n, B2n, V1_1[2 * H], dstg, srcg, src3, ea3)
    _, out_v = _node_update(
        vh, seg, cnt_v, lp1['vmsg2']['W'], lp1['vmsg2']['b'],
        lp1['vmlp1']['W'], lp1['vmlp1']['b'], lp1['vln_g'], lp1['vln_b'],
        lp1['vmlp2']['W'], lp1['vmlp2']['b'],
        [(params['vout']['W'], params['vout']['b'])])

    return out_c, out_v


# speedup vs baseline: 1.3061x; 1.3061x over previous
import functools

import jax
import jax.numpy as jnp
from jax import lax
from jax.experimental import pallas as pl
from jax.experimental.pallas import tpu as pltpu
from jax.experimental.pallas import tpu_sc as plsc

NN = 50000
EE = 800000
H = 64
HH = 32
NSUB = 16
KC = 128
SUP = 8
NSUP = 50
EPAD = NSUB * NSUP * SUP * KC
EPTP = EPAD // NSUB
NR3 = EPAD // (SUP * KC)
NACC = NN + 8
UR = 40
NUNITS = NN // UR

@functools.lru_cache(maxsize=None)
def _make_edge_seg_kernel():
    mesh = plsc.VectorSubcoreMesh(core_axis_name="c", subcore_axis_name="s")
    return functools.partial(
        pl.kernel,
        out_type=jax.ShapeDtypeStruct((2, NN, HH), jnp.float32),
        mesh=mesh,
        scratch_types=[
            pltpu.VMEM((SUP * KC,), jnp.int32),
            pltpu.VMEM((SUP * KC,), jnp.int32),
            pltpu.VMEM((SUP, KC), jnp.int32),
            pltpu.VMEM((SUP, KC), jnp.float32),
            pltpu.VMEM((16,), jnp.int32),
            pltpu.VMEM((KC, 16), jnp.int32),
            pltpu.VMEM((KC, 16), jnp.int32),
            pltpu.VMEM((KC, 16), jnp.int32),
            pltpu.VMEM((KC, 16), jnp.int32),
            pltpu.VMEM((KC, HH), jnp.float32),
            pltpu.VMEM((KC, HH), jnp.float32),
            pltpu.VMEM((UR, HH), jnp.float32),
            pltpu.VMEM_SHARED((NACC, HH), jnp.float32),
            pltpu.SemaphoreType.DMA,
            pltpu.SemaphoreType.DMA,
            pltpu.SemaphoreType.DMA,
            pltpu.SemaphoreType.DMA,
            pltpu.SemaphoreType.DMA,
            pltpu.SemaphoreType.DMA,
        ],
        compiler_params=pltpu.CompilerParams(needs_layout_passes=False, use_tc_tiling_on_sc=False),
    )(_edge_seg_body)


def _edge_seg_body(tD, tS, dstg, srcg, agg3, ea3, wa, out,
                   dbuf, sbuf, aggbuf, eabuf, wabuf, Ab0, Ab1, Bb0, Bb1,
                   Mb0, Mb1, Zb, acc, sga0, sga1, sgb0, sgb1, ssc0, ssc1):
    c = lax.axis_index("c")
    s = lax.axis_index("s")
    zero16 = jnp.zeros((16,), jnp.float32)
    Abs_ = (Ab0, Ab1)
    Bbs = (Bb0, Bb1)
    Mbs = (Mb0, Mb1)
    sgas = (sga0, sga1)
    sgbs = (sgb0, sgb1)
    sscs = (ssc0, ssc1)

    with jax.named_scope("zero_acc"):
        def zrow(i, carry):
            Zb[i, pl.ds(0, 16)] = zero16
            Zb[i, pl.ds(16, 16)] = zero16
            return carry
        lax.fori_loop(0, UR, zrow, 0)

        nunits = (NUNITS - s + NSUB - 1) // NSUB

        def zcp(i, carry):
            pltpu.sync_copy(Zb, acc.at[pl.ds((s + NSUB * i) * UR, UR)])
            return carry
        lax.fori_loop(0, nunits, zcp, 0)

        pltpu.sync_copy(wa.at[pl.ds(c * 16, 16)], wabuf)
        wab = plsc.bitcast(wabuf[pl.ds(0, 16)], jnp.bfloat16)
        plsc.subcore_barrier()

    def fire_gathers(k, b):
        sl = pl.ds(k * KC, KC)
        pltpu.async_copy(tD.at[dbuf.at[sl]], Abs_[b], sgas[b])
        pltpu.async_copy(tS.at[sbuf.at[sl]], Bbs[b], sgbs[b])

    def wait_gathers(k, b):
        sl = pl.ds(k * KC, KC)
        pltpu.make_async_copy(tD.at[dbuf.at[sl]], Abs_[b], sgas[b]).wait()
        pltpu.make_async_copy(tS.at[sbuf.at[sl]], Bbs[b], sgbs[b]).wait()

    def superbody(sp, carry):
        off = c * EPAD + s * EPTP + sp * (SUP * KC)
        r = s * NSUP + sp
        pltpu.sync_copy(dstg.at[pl.ds(off, SUP * KC)], dbuf)
        pltpu.sync_copy(srcg.at[pl.ds(off, SUP * KC)], sbuf)
        pltpu.sync_copy(agg3.at[r], aggbuf)
        pltpu.sync_copy(ea3.at[r], eabuf)
        fire_gathers(0, 0)

        def pairbody(p, carry2):
            for b in range(2):
                k = 2 * p + b
                kn = jnp.minimum(k + 1, SUP - 1)

                @pl.when(k < SUP - 1)
                def _():
                    fire_gathers(kn, 1 - b)

                wait_gathers(k, b)

                @pl.when(k >= 2)
                def _():
                    pltpu.make_async_copy(Mbs[b], acc.at[aggbuf.at[k]],
                                          sscs[b]).wait()

                Ab = Abs_[b]
                Bb = Bbs[b]
                Mb = Mbs[b]
                kvec = jnp.full((16,), k, jnp.int32)

                def _ld(e):
                    eb = plsc.load_gather(
                        eabuf, [kvec, jnp.full((16,), e, jnp.int32)])
                    a32 = plsc.bitcast(Ab[e, pl.ds(0, 16)], jnp.bfloat16)
                    b32 = plsc.bitcast(Bb[e, pl.ds(0, 16)], jnp.bfloat16)
                    return eb, a32, b32

                eb, a32, b32 = _ld(0)
                for e in range(KC):
                    nxt = _ld(e + 1) if e < KC - 1 else None
                    ebb = plsc.pack(eb, eb, format=plsc.PackFormat.INTERLEAVED)
                    m32 = jnp.maximum(a32 + b32 + ebb * wab, 0.0)
                    m0, m1 = plsc.unpack(m32,
                                         format=plsc.PackFormat.INTERLEAVED)
                    Mb[e, pl.ds(0, 16)] = m0
                    Mb[e, pl.ds(16, 16)] = m1
                    if nxt is not None:
                        eb, a32, b32 = nxt
                pltpu.async_copy(Mb, acc.at[aggbuf.at[k]], sscs[b], add=True)
            return carry2
        lax.fori_loop(0, SUP // 2, pairbody, 0)

        for b in range(2):
            pltpu.make_async_copy(Mbs[b], acc.at[aggbuf.at[SUP - 2 + b]],
                                  sscs[b]).wait()
        return carry
    with jax.named_scope("edge_loop"):
        lax.fori_loop(0, NSUP, superbody, 0)
        plsc.subcore_barrier()

    with jax.named_scope("copy_out"):
        def ocp(i, carry):
            rs = (s + NSUB * i) * UR
            pltpu.sync_copy(acc.at[pl.ds(rs, UR)], out.at[c, pl.ds(rs, UR)])
            return carry
        lax.fori_loop(0, nunits, ocp, 0)


_CW = 16


@functools.lru_cache(maxsize=None)
def _make_count_kernel():
    mesh = plsc.VectorSubcoreMesh(core_axis_name="c", subcore_axis_name="s")
    return functools.partial(
        pl.kernel,
        out_type=jax.ShapeDtypeStruct((2, NN, _CW), jnp.float32),
        mesh=mesh,
        scratch_types=[
            pltpu.VMEM((SUP, KC), jnp.int32),
            pltpu.VMEM((KC, _CW), jnp.float32),
            pltpu.VMEM((UR, _CW), jnp.float32),
            pltpu.VMEM_SHARED((NACC, _CW), jnp.float32),
        ],
        compiler_params=pltpu.CompilerParams(needs_layout_passes=False, use_tc_tiling_on_sc=False),
    )(_count_body)


def _count_body(agg4, out, aggbuf, Ob, Zb, acc):
    c = lax.axis_index("c")
    s = lax.axis_index("s")
    zero16 = jnp.zeros((16,), jnp.float32)
    one16 = jnp.ones((16,), jnp.float32)

    def fill(i, carry):
        Zb[i, pl.ds(0, 16)] = zero16
        return carry
    lax.fori_loop(0, UR, fill, 0)

    def fillo(i, carry):
        Ob[i, pl.ds(0, 16)] = one16
        return carry
    lax.fori_loop(0, KC, fillo, 0)

    nunits = (NUNITS - s + NSUB - 1) // NSUB

    def zcp(i, carry):
        pltpu.sync_copy(Zb, acc.at[pl.ds((s + NSUB * i) * UR, UR)])
        return carry
    lax.fori_loop(0, nunits, zcp, 0)
    plsc.subcore_barrier()

    def superbody(sp, carry):
        r = c * NR3 + s * NSUP + sp
        pltpu.sync_copy(agg4.at[r], aggbuf)

        def chunkbody(k, carry2):
            pltpu.sync_copy(Ob, acc.at[aggbuf.at[k]], add=True)
            return carry2
        lax.fori_loop(0, SUP, chunkbody, 0)
        return carry
    with jax.named_scope("edge_loop"):
        lax.fori_loop(0, NSUP, superbody, 0)
        plsc.subcore_barrier()

    with jax.named_scope("copy_out"):
        def ocp(i, carry):
            rs = (s + NSUB * i) * UR
            pltpu.sync_copy(acc.at[pl.ds(rs, UR)], out.at[c, pl.ds(rs, UR)])
            return carry
        lax.fori_loop(0, nunits, ocp, 0)


_RB = 1000
_NG = NN // _RB


def _full(shape):
    return pl.BlockSpec(shape, lambda i: (0,) * len(shape))


def _rows(width):
    return pl.BlockSpec((_RB, width), lambda i: (i, 0))


def _project(x, W, b, extras):
    din = x.shape[1]
    ne = len(extras)

    def body(*refs):
        x_ref, W_ref, b_ref = refs[:3]
        prefs = refs[3:3 + 2 * ne]
        outs = refs[3 + 2 * ne:]
        y = jnp.dot(x_ref[...], W_ref[...],
                    preferred_element_type=jnp.float32) + b_ref[...]
        outs[0][...] = y
        for t in range(ne):
            outs[1 + t][...] = jnp.dot(
                y, prefs[2 * t][...],
                preferred_element_type=jnp.float32) + prefs[2 * t + 1][...]

    in_specs = [_rows(din), _full((din, H)), _full((1, H))]
    args = [x, W, b.reshape(1, H)]
    for (P, p) in extras:
        in_specs += [_full((H, H)), _full((1, H))]
        args += [P, p.reshape(1, H)]
    out_shapes = tuple(jax.ShapeDtypeStruct((NN, H), jnp.float32)
                       for _ in range(1 + ne))
    out_specs = tuple(_rows(H) for _ in range(1 + ne))
    return pl.pallas_call(
        body, grid=(_NG,), in_specs=in_specs, out_specs=out_specs,
        out_shape=out_shapes)(*args)


def _node_update(x, seg, cnt, W2, b2, M1, bm1, g, bln, M2, bm2, extras):
    ne = len(extras)

    def body(*refs):
        (x_ref, s0_ref, s1_ref, cnt_ref, W2_ref, b2_ref, M1a_ref, M1b_ref,
         bm1_ref, g_ref, bln_ref, M2_ref, bm2_ref) = refs[:13]
        prefs = refs[13:13 + 2 * ne]
        outs = refs[13 + 2 * ne:]
        cntv = cnt_ref[...]
        seg_ = jnp.concatenate([s0_ref[...], s1_ref[...]], axis=1)
        msg = (jnp.dot(seg_ / jnp.maximum(cntv, 1.0), W2_ref[...],
                       preferred_element_type=jnp.float32)
               + b2_ref[...] * (cntv > 0))
        t = (jnp.dot(x_ref[...], M1a_ref[...],
                     preferred_element_type=jnp.float32)
             + jnp.dot(msg, M1b_ref[...], preferred_element_type=jnp.float32)
             + bm1_ref[...])
        mu = jnp.mean(t, axis=-1, keepdims=True)
        var = jnp.mean((t - mu) ** 2, axis=-1, keepdims=True)
        h = jnp.maximum(
            g_ref[...] * (t - mu) / jnp.sqrt(var + 1e-5) + bln_ref[...], 0.0)
        y = jnp.dot(h, M2_ref[...],
                    preferred_element_type=jnp.float32) + bm2_ref[...]
        outs[0][...] = y
        for k in range(ne):
            outs[1 + k][...] = jnp.dot(
                y, prefs[2 * k][...],
                preferred_element_type=jnp.float32) + prefs[2 * k + 1][...]

    in_specs = [_rows(H), _rows(HH), _rows(HH), pl.BlockSpec((_RB, 1), lambda i: (i, 0)),
                _full((H, H)), _full((1, H)), _full((H, H)), _full((H, H)),
                _full((1, H)), _full((1, H)), _full((1, H)), _full((H, H)),
                _full((1, H))]
    args = [x, seg[0], seg[1], cnt, W2, b2.reshape(1, H), M1[:H], M1[H:],
            bm1.reshape(1, H), g.reshape(1, H), bln.reshape(1, H), M2,
            bm2.reshape(1, H)]
    for (P, p) in extras:
        in_specs += [_full((H, H)), _full((1, H))]
        args += [P, p.reshape(1, H)]
    out_shapes = tuple(jax.ShapeDtypeStruct((NN, H), jnp.float32)
                       for _ in range(1 + ne))
    out_specs = tuple(_rows(H) for _ in range(1 + ne))
    return pl.pallas_call(
        body, grid=(_NG,), in_specs=in_specs, out_specs=out_specs,
        out_shape=out_shapes)(*args)


def _pack_half(t32):
    lo = lax.bitcast_convert_type(t32[:, :16].astype(jnp.bfloat16),
                                  jnp.uint16).astype(jnp.uint32)
    hi = lax.bitcast_convert_type(t32[:, 16:].astype(jnp.bfloat16),
                                  jnp.uint16).astype(jnp.uint32)
    return (lo | (hi << 16)).astype(jnp.int32)


def _pack_table(t):
    return jnp.concatenate([_pack_half(t[:, :HH]), _pack_half(t[:, HH:])],
                           axis=0)


def _edge_phase(tableD, tableS, wa, dstg, srcg, agg3, ea3):
    wa_p = jnp.concatenate([_pack_half(wa[None, :HH]),
                            _pack_half(wa[None, HH:])], axis=0).reshape(-1)
    seg2 = _make_edge_seg_kernel()(_pack_table(tableD), _pack_table(tableS),
                                   dstg, srcg, agg3, ea3, wa_p)
    return seg2


def kernel(constraint_features, variable_features, edge_index, edge_attr,
           params):
    src = edge_index[0].astype(jnp.int32)
    dst = edge_index[1].astype(jnp.int32)
    ea = edge_attr.astype(jnp.float32)

    npad = EPAD - EE
    zpad = jnp.zeros((npad,), jnp.int32)
    dstp = jnp.concatenate([dst, zpad])
    srcp = jnp.concatenate([src, zpad])
    dsta = jnp.concatenate([dst, jnp.full((npad,), NN, jnp.int32)])
    srca = jnp.concatenate([src, jnp.full((npad,), NN, jnp.int32)])
    eap = jnp.concatenate([ea, jnp.zeros((npad,), jnp.float32)])
    dstg = jnp.concatenate([dstp, dstp + NN])
    srcg = jnp.concatenate([srcp, srcp + NN])
    dst3 = dsta.reshape(NR3, SUP, KC)
    src3 = srca.reshape(NR3, SUP, KC)
    ea3 = eap.reshape(NR3, SUP, KC)

    cnts = _make_count_kernel()(
        jnp.concatenate([dst3, src3]))
    cnt_c = cnts[0, :, 0:1]
    cnt_v = cnts[1, :, 0:1]

    lp0, lp1 = params['layers']
    W1_0 = lp0['cmsg1']['W']
    V1_0 = lp0['vmsg1']['W']
    W1_1 = lp1['cmsg1']['W']
    V1_1 = lp1['vmsg1']['W']

    cf = jnp.pad(constraint_features, ((0, 0), (0, 3)))
    vf = jnp.pad(variable_features, ((0, 0), (0, 5)))
    Wc = jnp.pad(params['cin']['W'], ((0, 3), (0, 0)))
    Wv = jnp.pad(params['vin']['W'], ((0, 5), (0, 0)))
    ch, A1 = _project(cf, Wc, params['cin']['b'],
                      [(W1_0[:H], jnp.zeros((H,), jnp.float32))])
    vh, B1, B2 = _project(
        vf, Wv, params['vin']['b'],
        [(W1_0[H:2 * H], lp0['cmsg1']['b']),
         (V1_0[:H], lp0['vmsg1']['b'])])

    seg = _edge_phase(A1, B1, W1_0[2 * H], dstg, srcg, dst3, ea3)
    ch, A2, A1n = _node_update(
        ch, seg, cnt_c, lp0['cmsg2']['W'], lp0['cmsg2']['b'],
        lp0['cmlp1']['W'], lp0['cmlp1']['b'], lp0['cln_g'], lp0['cln_b'],
        lp0['cmlp2']['W'], lp0['cmlp2']['b'],
        [(V1_0[H:2 * H], jnp.zeros((H,), jnp.float32)),
         (W1_1[:H], jnp.zeros((H,), jnp.float32))])

    seg = _edge_phase(A2, B2, V1_0[2 * H], dstg, srcg, src3, ea3)
    vh, B1n, B2n = _node_update(
        vh, seg, cnt_v, lp0['vmsg2']['W'], lp0['vmsg2']['b'],
        lp0['vmlp1']['W'], lp0['vmlp1']['b'], lp0['vln_g'], lp0['vln_b'],
        lp0['vmlp2']['W'], lp0['vmlp2']['b'],
        [(W1_1[H:2 * H], lp1['cmsg1']['b']),
         (V1_1[:H], lp1['vmsg1']['b'])])

    seg = _edge_phase(A1n, B1n, W1_1[2 * H], dstg, srcg, dst3, ea3)
    ch, A2n, out_c = _node_update(
        ch, seg, cnt_c, lp1['cmsg2']['W'], lp1['cmsg2']['b'],
        lp1['cmlp1']['W'], lp1['cmlp1']['b'], lp1['cln_g'], lp1['cln_b'],
        lp1['cmlp2']['W'], lp1['cmlp2']['b'],
        [(V1_1[H:2 * H], jnp.zeros((H,), jnp.float32)),
         (params['cout']['W'], params['cout']['b'])])

    seg = _edge_phase(A2n, B2n, V1_1[2 * H], dstg, srcg, src3, ea3)
    _, out_v = _node_update(
        vh, seg, cnt_v, lp1['vmsg2']['W'], lp1['vmsg2']['b'],
        lp1['vmlp1']['W'], lp1['vmlp1']['b'], lp1['vln_g'], lp1['vln_b'],
        lp1['vmlp2']['W'], lp1['vmlp2']['b'],
        [(params['vout']['W'], params['vout']['b'])])

    return out_c, out_v


# speedup vs baseline: 1.4709x; 1.1261x over previous
import functools

import jax
import jax.numpy as jnp
from jax import lax
from jax.experimental import pallas as pl
from jax.experimental.pallas import tpu as pltpu
from jax.experimental.pallas import tpu_sc as plsc

NN = 50000
EE = 800000
H = 64
HH = 32
NSUB = 16
KC = 128
SUP = 8
NSUP = 50
EPAD = NSUB * NSUP * SUP * KC
EPTP = EPAD // NSUB
NR3 = EPAD // (SUP * KC)
NACC = NN + 8
UR = 40
NUNITS = NN // UR

@functools.lru_cache(maxsize=None)
def _make_edge_seg_kernel():
    mesh = plsc.VectorSubcoreMesh(core_axis_name="c", subcore_axis_name="s")
    return functools.partial(
        pl.kernel,
        out_type=jax.ShapeDtypeStruct((2, NN, HH), jnp.bfloat16),
        mesh=mesh,
        scratch_types=[
            pltpu.VMEM((SUP * KC,), jnp.int32),
            pltpu.VMEM((SUP * KC,), jnp.int32),
            pltpu.VMEM((SUP, KC), jnp.int32),
            pltpu.VMEM((SUP, KC), jnp.float32),
            pltpu.VMEM((16,), jnp.int32),
            pltpu.VMEM((KC, 16), jnp.int32),
            pltpu.VMEM((KC, 16), jnp.int32),
            pltpu.VMEM((KC, 16), jnp.int32),
            pltpu.VMEM((KC, 16), jnp.int32),
            pltpu.VMEM((KC, HH), jnp.bfloat16),
            pltpu.VMEM((KC, HH), jnp.bfloat16),
            pltpu.VMEM((UR, HH), jnp.bfloat16),
            pltpu.VMEM_SHARED((NACC, HH), jnp.bfloat16),
            pltpu.SemaphoreType.DMA,
            pltpu.SemaphoreType.DMA,
            pltpu.SemaphoreType.DMA,
            pltpu.SemaphoreType.DMA,
            pltpu.SemaphoreType.DMA,
            pltpu.SemaphoreType.DMA,
        ],
        compiler_params=pltpu.CompilerParams(needs_layout_passes=False, use_tc_tiling_on_sc=False),
    )(_edge_seg_body)


def _edge_seg_body(tD, tS, dstg, srcg, agg3, ea3, wa, out,
                   dbuf, sbuf, aggbuf, eabuf, wabuf, Ab0, Ab1, Bb0, Bb1,
                   Mb0, Mb1, Zb, acc, sga0, sga1, sgb0, sgb1, ssc0, ssc1):
    c = lax.axis_index("c")
    s = lax.axis_index("s")
    zero16 = jnp.zeros((16,), jnp.float32)
    Abs_ = (Ab0, Ab1)
    Bbs = (Bb0, Bb1)
    Mbs = (Mb0, Mb1)
    sgas = (sga0, sga1)
    sgbs = (sgb0, sgb1)
    sscs = (ssc0, ssc1)

    with jax.named_scope("zero_acc"):
        zero32b = jnp.zeros((HH,), jnp.bfloat16)

        def zrow(i, carry):
            Zb[i, :] = zero32b
            return carry
        lax.fori_loop(0, UR, zrow, 0)

        nunits = (NUNITS - s + NSUB - 1) // NSUB

        def zcp(i, carry):
            pltpu.sync_copy(Zb, acc.at[pl.ds((s + NSUB * i) * UR, UR)])
            return carry
        lax.fori_loop(0, nunits, zcp, 0)

        pltpu.sync_copy(wa.at[pl.ds(c * 16, 16)], wabuf)
        wab = plsc.bitcast(wabuf[pl.ds(0, 16)], jnp.bfloat16)
        plsc.subcore_barrier()

    def fire_gathers(k, b):
        sl = pl.ds(k * KC, KC)
        pltpu.async_copy(tD.at[dbuf.at[sl]], Abs_[b], sgas[b])
        pltpu.async_copy(tS.at[sbuf.at[sl]], Bbs[b], sgbs[b])

    def wait_gathers(k, b):
        sl = pl.ds(k * KC, KC)
        pltpu.make_async_copy(tD.at[dbuf.at[sl]], Abs_[b], sgas[b]).wait()
        pltpu.make_async_copy(tS.at[sbuf.at[sl]], Bbs[b], sgbs[b]).wait()

    def superbody(sp, carry):
        off = c * EPAD + s * EPTP + sp * (SUP * KC)
        r = s * NSUP + sp
        pltpu.sync_copy(dstg.at[pl.ds(off, SUP * KC)], dbuf)
        pltpu.sync_copy(srcg.at[pl.ds(off, SUP * KC)], sbuf)
        pltpu.sync_copy(agg3.at[r], aggbuf)
        pltpu.sync_copy(ea3.at[r], eabuf)
        fire_gathers(0, 0)

        def pairbody(p, carry2):
            for b in range(2):
                k = 2 * p + b
                kn = jnp.minimum(k + 1, SUP - 1)

                @pl.when(k < SUP - 1)
                def _():
                    fire_gathers(kn, 1 - b)

                wait_gathers(k, b)

                @pl.when(k >= 2)
                def _():
                    pltpu.make_async_copy(Mbs[b], acc.at[aggbuf.at[k]],
                                          sscs[b]).wait()

                Ab = Abs_[b]
                Bb = Bbs[b]
                Mb = Mbs[b]
                kvec = jnp.full((16,), k, jnp.int32)

                def _ld(e):
                    eb = plsc.load_gather(
                        eabuf, [kvec, jnp.full((16,), e, jnp.int32)])
                    a32 = plsc.bitcast(Ab[e, pl.ds(0, 16)], jnp.bfloat16)
                    b32 = plsc.bitcast(Bb[e, pl.ds(0, 16)], jnp.bfloat16)
                    return eb, a32, b32

                eb, a32, b32 = _ld(0)
                for e in range(KC):
                    nxt = _ld(e + 1) if e < KC - 1 else None
                    ebb = plsc.pack(eb, eb, format=plsc.PackFormat.INTERLEAVED)
                    m32 = jnp.maximum(a32 + b32 + ebb * wab, 0.0)
                    Mb[e, :] = m32
                    if nxt is not None:
                        eb, a32, b32 = nxt
                pltpu.async_copy(Mb, acc.at[aggbuf.at[k]], sscs[b], add=True)
            return carry2
        lax.fori_loop(0, SUP // 2, pairbody, 0)

        for b in range(2):
            pltpu.make_async_copy(Mbs[b], acc.at[aggbuf.at[SUP - 2 + b]],
                                  sscs[b]).wait()
        return carry
    with jax.named_scope("edge_loop"):
        lax.fori_loop(0, NSUP, superbody, 0)
        plsc.subcore_barrier()

    with jax.named_scope("copy_out"):
        def ocp(i, carry):
            rs = (s + NSUB * i) * UR
            pltpu.sync_copy(acc.at[pl.ds(rs, UR)], out.at[c, pl.ds(rs, UR)])
            return carry
        lax.fori_loop(0, nunits, ocp, 0)


_CW = 16


@functools.lru_cache(maxsize=None)
def _make_count_kernel():
    mesh = plsc.VectorSubcoreMesh(core_axis_name="c", subcore_axis_name="s")
    return functools.partial(
        pl.kernel,
        out_type=jax.ShapeDtypeStruct((2, NN, _CW), jnp.float32),
        mesh=mesh,
        scratch_types=[
            pltpu.VMEM((SUP, KC), jnp.int32),
            pltpu.VMEM((KC, _CW), jnp.float32),
            pltpu.VMEM((UR, _CW), jnp.float32),
            pltpu.VMEM_SHARED((NACC, _CW), jnp.float32),
        ],
        compiler_params=pltpu.CompilerParams(needs_layout_passes=False, use_tc_tiling_on_sc=False),
    )(_count_body)


def _count_body(agg4, out, aggbuf, Ob, Zb, acc):
    c = lax.axis_index("c")
    s = lax.axis_index("s")
    zero16 = jnp.zeros((16,), jnp.float32)
    one16 = jnp.ones((16,), jnp.float32)

    def fill(i, carry):
        Zb[i, pl.ds(0, 16)] = zero16
        return carry
    lax.fori_loop(0, UR, fill, 0)

    def fillo(i, carry):
        Ob[i, pl.ds(0, 16)] = one16
        return carry
    lax.fori_loop(0, KC, fillo, 0)

    nunits = (NUNITS - s + NSUB - 1) // NSUB

    def zcp(i, carry):
        pltpu.sync_copy(Zb, acc.at[pl.ds((s + NSUB * i) * UR, UR)])
        return carry
    lax.fori_loop(0, nunits, zcp, 0)
    plsc.subcore_barrier()

    def superbody(sp, carry):
        r = c * NR3 + s * NSUP + sp
        pltpu.sync_copy(agg4.at[r], aggbuf)

        def chunkbody(k, carry2):
            pltpu.sync_copy(Ob, acc.at[aggbuf.at[k]], add=True)
            return carry2
        lax.fori_loop(0, SUP, chunkbody, 0)
        return carry
    with jax.named_scope("edge_loop"):
        lax.fori_loop(0, NSUP, superbody, 0)
        plsc.subcore_barrier()

    with jax.named_scope("copy_out"):
        def ocp(i, carry):
            rs = (s + NSUB * i) * UR
            pltpu.sync_copy(acc.at[pl.ds(rs, UR)], out.at[c, pl.ds(rs, UR)])
            return carry
        lax.fori_loop(0, nunits, ocp, 0)


_RB = 1000
_NG = NN // _RB


def _full(shape):
    return pl.BlockSpec(shape, lambda i: (0,) * len(shape))


def _rows(width):
    return pl.BlockSpec((_RB, width), lambda i: (i, 0))


def _project(x, W, b, extras):
    din = x.shape[1]
    ne = len(extras)

    def body(*refs):
        x_ref, W_ref, b_ref = refs[:3]
        prefs = refs[3:3 + 2 * ne]
        outs = refs[3 + 2 * ne:]
        y = jnp.dot(x_ref[...], W_ref[...],
                    preferred_element_type=jnp.float32) + b_ref[...]
        outs[0][...] = y
        for t in range(ne):
            outs[1 + t][...] = jnp.dot(
                y, prefs[2 * t][...],
                preferred_element_type=jnp.float32) + prefs[2 * t + 1][...]

    in_specs = [_rows(din), _full((din, H)), _full((1, H))]
    args = [x, W, b.reshape(1, H)]
    for (P, p) in extras:
        in_specs += [_full((H, H)), _full((1, H))]
        args += [P, p.reshape(1, H)]
    out_shapes = tuple(jax.ShapeDtypeStruct((NN, H), jnp.float32)
                       for _ in range(1 + ne))
    out_specs = tuple(_rows(H) for _ in range(1 + ne))
    return pl.pallas_call(
        body, grid=(_NG,), in_specs=in_specs, out_specs=out_specs,
        out_shape=out_shapes)(*args)


def _node_update(x, seg, cnt, W2, b2, M1, bm1, g, bln, M2, bm2, extras):
    ne = len(extras)

    def body(*refs):
        (x_ref, s0_ref, s1_ref, cnt_ref, W2_ref, b2_ref, M1a_ref, M1b_ref,
         bm1_ref, g_ref, bln_ref, M2_ref, bm2_ref) = refs[:13]
        prefs = refs[13:13 + 2 * ne]
        outs = refs[13 + 2 * ne:]
        cntv = cnt_ref[...]
        seg_ = jnp.concatenate([s0_ref[...], s1_ref[...]],
                               axis=1).astype(jnp.float32)
        msg = (jnp.dot(seg_ / jnp.maximum(cntv, 1.0), W2_ref[...],
                       preferred_element_type=jnp.float32)
               + b2_ref[...] * (cntv > 0))
        t = (jnp.dot(x_ref[...], M1a_ref[...],
                     preferred_element_type=jnp.float32)
             + jnp.dot(msg, M1b_ref[...], preferred_element_type=jnp.float32)
             + bm1_ref[...])
        mu = jnp.mean(t, axis=-1, keepdims=True)
        var = jnp.mean((t - mu) ** 2, axis=-1, keepdims=True)
        h = jnp.maximum(
            g_ref[...] * (t - mu) / jnp.sqrt(var + 1e-5) + bln_ref[...], 0.0)
        y = jnp.dot(h, M2_ref[...],
                    preferred_element_type=jnp.float32) + bm2_ref[...]
        outs[0][...] = y
        for k in range(ne):
            outs[1 + k][...] = jnp.dot(
                y, prefs[2 * k][...],
                preferred_element_type=jnp.float32) + prefs[2 * k + 1][...]

    in_specs = [_rows(H), _rows(HH), _rows(HH), pl.BlockSpec((_RB, 1), lambda i: (i, 0)),
                _full((H, H)), _full((1, H)), _full((H, H)), _full((H, H)),
                _full((1, H)), _full((1, H)), _full((1, H)), _full((H, H)),
                _full((1, H))]
    args = [x, seg[0], seg[1], cnt, W2, b2.reshape(1, H), M1[:H], M1[H:],
            bm1.reshape(1, H), g.reshape(1, H), bln.reshape(1, H), M2,
            bm2.reshape(1, H)]
    for (P, p) in extras:
        in_specs += [_full((H, H)), _full((1, H))]
        args += [P, p.reshape(1, H)]
    out_shapes = tuple(jax.ShapeDtypeStruct((NN, H), jnp.float32)
                       for _ in range(1 + ne))
    out_specs = tuple(_rows(H) for _ in range(1 + ne))
    return pl.pallas_call(
        body, grid=(_NG,), in_specs=in_specs, out_specs=out_specs,
        out_shape=out_shapes)(*args)


def _pack_half(t32):
    lo = lax.bitcast_convert_type(t32[:, :16].astype(jnp.bfloat16),
                                  jnp.uint16).astype(jnp.uint32)
    hi = lax.bitcast_convert_type(t32[:, 16:].astype(jnp.bfloat16),
                                  jnp.uint16).astype(jnp.uint32)
    return (lo | (hi << 16)).astype(jnp.int32)


def _pack_table(t):
    return jnp.concatenate([_pack_half(t[:, :HH]), _pack_half(t[:, HH:])],
                           axis=0)


def _edge_phase(tableD, tableS, wa, dstg, srcg, agg3, ea3):
    wa_p = jnp.concatenate([_pack_half(wa[None, :HH]),
                            _pack_half(wa[None, HH:])], axis=0).reshape(-1)
    seg2 = _make_edge_seg_kernel()(_pack_table(tableD), _pack_table(tableS),
                                   dstg, srcg, agg3, ea3, wa_p)
    s3 = seg2.reshape(2, NN, 16, 2)
    return jnp.concatenate([s3[..., 0], s3[..., 1]], axis=2)


def kernel(constraint_features, variable_features, edge_index, edge_attr,
           params):
    src = edge_index[0].astype(jnp.int32)
    dst = edge_index[1].astype(jnp.int32)
    ea = edge_attr.astype(jnp.float32)

    npad = EPAD - EE
    zpad = jnp.zeros((npad,), jnp.int32)
    dstp = jnp.concatenate([dst, zpad])
    srcp = jnp.concatenate([src, zpad])
    dsta = jnp.concatenate([dst, jnp.full((npad,), NN, jnp.int32)])
    srca = jnp.concatenate([src, jnp.full((npad,), NN, jnp.int32)])
    eap = jnp.concatenate([ea, jnp.zeros((npad,), jnp.float32)])
    dstg = jnp.concatenate([dstp, dstp + NN])
    srcg = jnp.concatenate([srcp, srcp + NN])
    dst3 = dsta.reshape(NR3, SUP, KC)
    src3 = srca.reshape(NR3, SUP, KC)
    ea3 = eap.reshape(NR3, SUP, KC)

    cnts = _make_count_kernel()(
        jnp.concatenate([dst3, src3]))
    cnt_c = cnts[0, :, 0:1]
    cnt_v = cnts[1, :, 0:1]

    lp0, lp1 = params['layers']
    W1_0 = lp0['cmsg1']['W']
    V1_0 = lp0['vmsg1']['W']
    W1_1 = lp1['cmsg1']['W']
    V1_1 = lp1['vmsg1']['W']

    cf = jnp.pad(constraint_features, ((0, 0), (0, 3)))
    vf = jnp.pad(variable_features, ((0, 0), (0, 5)))
    Wc = jnp.pad(params['cin']['W'], ((0, 3), (0, 0)))
    Wv = jnp.pad(params['vin']['W'], ((0, 5), (0, 0)))
    ch, A1 = _project(cf, Wc, params['cin']['b'],
                      [(W1_0[:H], jnp.zeros((H,), jnp.float32))])
    vh, B1, B2 = _project(
        vf, Wv, params['vin']['b'],
        [(W1_0[H:2 * H], lp0['cmsg1']['b']),
         (V1_0[:H], lp0['vmsg1']['b'])])

    seg = _edge_phase(A1, B1, W1_0[2 * H], dstg, srcg, dst3, ea3)
    ch, A2, A1n = _node_update(
        ch, seg, cnt_c, lp0['cmsg2']['W'], lp0['cmsg2']['b'],
        lp0['cmlp1']['W'], lp0['cmlp1']['b'], lp0['cln_g'], lp0['cln_b'],
        lp0['cmlp2']['W'], lp0['cmlp2']['b'],
        [(V1_0[H:2 * H], jnp.zeros((H,), jnp.float32)),
         (W1_1[:H], jnp.zeros((H,), jnp.float32))])

    seg = _edge_phase(A2, B2, V1_0[2 * H], dstg, srcg, src3, ea3)
    vh, B1n, B2n = _node_update(
        vh, seg, cnt_v, lp0['vmsg2']['W'], lp0['vmsg2']['b'],
        lp0['vmlp1']['W'], lp0['vmlp1']['b'], lp0['vln_g'], lp0['vln_b'],
        lp0['vmlp2']['W'], lp0['vmlp2']['b'],
        [(W1_1[H:2 * H], lp1['cmsg1']['b']),
         (V1_1[:H], lp1['vmsg1']['b'])])

    seg = _edge_phase(A1n, B1n, W1_1[2 * H], dstg, srcg, dst3, ea3)
    ch, A2n, out_c = _node_update(
        ch, seg, cnt_c, lp1['cmsg2']['W'], lp1['cmsg2']['b'],
        lp1['cmlp1']['W'], lp1['cmlp1']['b'], lp1['cln_g'], lp1['cln_b'],
        lp1['cmlp2']['W'], lp1['cmlp2']['b'],
        [(V1_1[H:2 * H], jnp.zeros((H,), jnp.float32)),
         (params['cout']['W'], params['cout']['b'])])

    seg = _edge_phase(A2n, B2n, V1_1[2 * H], dstg, srcg, src3, ea3)
    _, out_v = _node_update(
        vh, seg, cnt_v, lp1['vmsg2']['W'], lp1['vmsg2']['b'],
        lp1['vmlp1']['W'], lp1['vmlp1']['b'], lp1['vln_g'], lp1['vln_b'],
        lp1['vmlp2']['W'], lp1['vmlp2']['b'],
        [(params['vout']['W'], params['vout']['b'])])

    return out_c, out_v


# speedup vs baseline: 1.6186x; 1.1004x over previous
import functools

import jax
import jax.numpy as jnp
from jax import lax
from jax.experimental import pallas as pl
from jax.experimental.pallas import tpu as pltpu
from jax.experimental.pallas import tpu_sc as plsc

NN = 50000
EE = 800000
H = 64
HH = 32
NSUB = 16
KC = 128
SUP = 8
NSUP = 50
EPAD = NSUB * NSUP * SUP * KC
EPTP = EPAD // NSUB
NR3 = EPAD // (SUP * KC)
NACC = NN + 8
UR = 40
NUNITS = NN // UR

@functools.lru_cache(maxsize=None)
def _make_edge_seg_kernel():
    mesh = plsc.VectorSubcoreMesh(core_axis_name="c", subcore_axis_name="s")
    return functools.partial(
        pl.kernel,
        out_type=jax.ShapeDtypeStruct((2, NN, HH), jnp.bfloat16),
        mesh=mesh,
        scratch_types=[
            pltpu.VMEM((SUP * KC,), jnp.int32),
            pltpu.VMEM((SUP * KC,), jnp.int32),
            pltpu.VMEM((SUP, KC), jnp.int32),
            pltpu.VMEM((SUP, KC), jnp.float32),
            pltpu.VMEM((16,), jnp.int32),
            pltpu.VMEM((KC, 16), jnp.int32),
            pltpu.VMEM((KC, 16), jnp.int32),
            pltpu.VMEM((KC, 16), jnp.int32),
            pltpu.VMEM((KC, 16), jnp.int32),
            pltpu.VMEM((KC, HH), jnp.bfloat16),
            pltpu.VMEM((KC, HH), jnp.bfloat16),
            pltpu.VMEM((UR, HH), jnp.bfloat16),
            pltpu.VMEM_SHARED((NACC, HH), jnp.bfloat16),
            pltpu.SemaphoreType.DMA,
            pltpu.SemaphoreType.DMA,
            pltpu.SemaphoreType.DMA,
            pltpu.SemaphoreType.DMA,
            pltpu.SemaphoreType.DMA,
            pltpu.SemaphoreType.DMA,
        ],
        compiler_params=pltpu.CompilerParams(needs_layout_passes=False, use_tc_tiling_on_sc=False),
    )(_edge_seg_body)


def _edge_seg_body(tD, tS, dstg, srcg, agg3, ea3, wa, out,
                   dbuf, sbuf, aggbuf, eabuf, wabuf, Ab0, Ab1, Bb0, Bb1,
                   Mb0, Mb1, Zb, acc, sga0, sga1, sgb0, sgb1, ssc0, ssc1):
    c = lax.axis_index("c")
    s = lax.axis_index("s")
    zero16 = jnp.zeros((16,), jnp.float32)
    Abs_ = (Ab0, Ab1)
    Bbs = (Bb0, Bb1)
    Mbs = (Mb0, Mb1)
    sgas = (sga0, sga1)
    sgbs = (sgb0, sgb1)
    sscs = (ssc0, ssc1)

    with jax.named_scope("zero_acc"):
        zero32b = jnp.zeros((HH,), jnp.bfloat16)

        def zrow(i, carry):
            Zb[i, :] = zero32b
            return carry
        lax.fori_loop(0, UR, zrow, 0)

        nunits = (NUNITS - s + NSUB - 1) // NSUB

        def zcp(i, carry):
            pltpu.sync_copy(Zb, acc.at[pl.ds((s + NSUB * i) * UR, UR)])
            return carry
        lax.fori_loop(0, nunits, zcp, 0)

        pltpu.sync_copy(wa.at[pl.ds(c * 16, 16)], wabuf)
        wab = plsc.bitcast(wabuf[pl.ds(0, 16)], jnp.bfloat16)
        plsc.subcore_barrier()

    def fire_gathers(k, b):
        sl = pl.ds(k * KC, KC)
        pltpu.async_copy(tD.at[dbuf.at[sl]], Abs_[b], sgas[b])
        pltpu.async_copy(tS.at[sbuf.at[sl]], Bbs[b], sgbs[b])

    def wait_gathers(k, b):
        sl = pl.ds(k * KC, KC)
        pltpu.make_async_copy(tD.at[dbuf.at[sl]], Abs_[b], sgas[b]).wait()
        pltpu.make_async_copy(tS.at[sbuf.at[sl]], Bbs[b], sgbs[b]).wait()

    def superbody(sp, carry):
        off = c * EPAD + s * EPTP + sp * (SUP * KC)
        r = s * NSUP + sp
        pltpu.sync_copy(dstg.at[pl.ds(off, SUP * KC)], dbuf)
        pltpu.sync_copy(srcg.at[pl.ds(off, SUP * KC)], sbuf)
        pltpu.sync_copy(agg3.at[r], aggbuf)
        pltpu.sync_copy(ea3.at[r], eabuf)
        fire_gathers(0, 0)

        def pairbody(p, carry2):
            for b in range(2):
                k = 2 * p + b
                kn = jnp.minimum(k + 1, SUP - 1)

                @pl.when(k < SUP - 1)
                def _():
                    fire_gathers(kn, 1 - b)

                wait_gathers(k, b)

                @pl.when(k >= 2)
                def _():
                    pltpu.make_async_copy(Mbs[b], acc.at[aggbuf.at[k]],
                                          sscs[b]).wait()

                Ab = Abs_[b]
                Bb = Bbs[b]
                Mb = Mbs[b]
                kvec = jnp.full((16,), k, jnp.int32)

                def _ld(e):
                    eb = plsc.load_gather(
                        eabuf, [kvec, jnp.full((16,), e, jnp.int32)])
                    a32 = plsc.bitcast(Ab[e, pl.ds(0, 16)], jnp.bfloat16)
                    b32 = plsc.bitcast(Bb[e, pl.ds(0, 16)], jnp.bfloat16)
                    return eb, a32, b32

                eb, a32, b32 = _ld(0)
                for e in range(KC):
                    nxt = _ld(e + 1) if e < KC - 1 else None
                    ebb = plsc.pack(eb, eb, format=plsc.PackFormat.INTERLEAVED)
                    m32 = jnp.maximum(a32 + b32 + ebb * wab, 0.0)
                    Mb[e, :] = m32
                    if nxt is not None:
                        eb, a32, b32 = nxt
                pltpu.async_copy(Mb, acc.at[aggbuf.at[k]], sscs[b], add=True)
            return carry2
        lax.fori_loop(0, SUP // 2, pairbody, 0)

        for b in range(2):
            pltpu.make_async_copy(Mbs[b], acc.at[aggbuf.at[SUP - 2 + b]],
                                  sscs[b]).wait()
        return carry
    with jax.named_scope("edge_loop"):
        lax.fori_loop(0, NSUP, superbody, 0)
        plsc.subcore_barrier()

    with jax.named_scope("copy_out"):
        def ocp(i, carry):
            rs = (s + NSUB * i) * UR
            pltpu.sync_copy(acc.at[pl.ds(rs, UR)], out.at[c, pl.ds(rs, UR)])
            return carry
        lax.fori_loop(0, nunits, ocp, 0)


_CW = 16


@functools.lru_cache(maxsize=None)
def _make_count_kernel():
    mesh = plsc.VectorSubcoreMesh(core_axis_name="c", subcore_axis_name="s")
    return functools.partial(
        pl.kernel,
        out_type=jax.ShapeDtypeStruct((2, NN, _CW), jnp.float32),
        mesh=mesh,
        scratch_types=[
            pltpu.VMEM((SUP, KC), jnp.int32),
            pltpu.VMEM((KC, _CW), jnp.float32),
            pltpu.VMEM((UR, _CW), jnp.float32),
            pltpu.VMEM_SHARED((NACC, _CW), jnp.float32),
        ],
        compiler_params=pltpu.CompilerParams(needs_layout_passes=False, use_tc_tiling_on_sc=False),
    )(_count_body)


def _count_body(agg4, out, aggbuf, Ob, Zb, acc):
    c = lax.axis_index("c")
    s = lax.axis_index("s")
    zero16 = jnp.zeros((16,), jnp.float32)
    one16 = jnp.ones((16,), jnp.float32)

    def fill(i, carry):
        Zb[i, pl.ds(0, 16)] = zero16
        return carry
    lax.fori_loop(0, UR, fill, 0)

    def fillo(i, carry):
        Ob[i, pl.ds(0, 16)] = one16
        return carry
    lax.fori_loop(0, KC, fillo, 0)

    nunits = (NUNITS - s + NSUB - 1) // NSUB

    def zcp(i, carry):
        pltpu.sync_copy(Zb, acc.at[pl.ds((s + NSUB * i) * UR, UR)])
        return carry
    lax.fori_loop(0, nunits, zcp, 0)
    plsc.subcore_barrier()

    def superbody(sp, carry):
        r = c * NR3 + s * NSUP + sp
        pltpu.sync_copy(agg4.at[r], aggbuf)

        def chunkbody(k, carry2):
            pltpu.sync_copy(Ob, acc.at[aggbuf.at[k]], add=True)
            return carry2
        lax.fori_loop(0, SUP, chunkbody, 0)
        return carry
    with jax.named_scope("edge_loop"):
        lax.fori_loop(0, NSUP, superbody, 0)
        plsc.subcore_barrier()

    with jax.named_scope("copy_out"):
        def ocp(i, carry):
            rs = (s + NSUB * i) * UR
            pltpu.sync_copy(acc.at[pl.ds(rs, UR)], out.at[c, pl.ds(rs, UR)])
            return carry
        lax.fori_loop(0, nunits, ocp, 0)


_RB = 1000
_NG = NN // _RB


def _full(shape):
    return pl.BlockSpec(shape, lambda i: (0,) * len(shape))


def _rows(width):
    return pl.BlockSpec((_RB, width), lambda i: (i, 0))


def _project(x, W, b, extras):
    din = x.shape[1]
    ne = len(extras)

    def body(*refs):
        x_ref, W_ref, b_ref = refs[:3]
        prefs = refs[3:3 + 2 * ne]
        outs = refs[3 + 2 * ne:]
        y = jnp.dot(x_ref[...], W_ref[...],
                    preferred_element_type=jnp.float32) + b_ref[...]
        outs[0][...] = y
        for t in range(ne):
            outs[1 + t][...] = jnp.dot(
                y, prefs[2 * t][...],
                preferred_element_type=jnp.float32) + prefs[2 * t + 1][...]

    in_specs = [_rows(din), _full((din, H)), _full((1, H))]
    args = [x, W, b.reshape(1, H)]
    for (P, p) in extras:
        in_specs += [_full((H, H)), _full((1, H))]
        args += [P, p.reshape(1, H)]
    out_shapes = tuple(jax.ShapeDtypeStruct((NN, H), jnp.float32)
                       for _ in range(1 + ne))
    out_specs = tuple(_rows(H) for _ in range(1 + ne))
    return pl.pallas_call(
        body, grid=(_NG,), in_specs=in_specs, out_specs=out_specs,
        out_shape=out_shapes)(*args)


def _node_update(x, seg, cnt, W2, b2, M1, bm1, g, bln, M2, bm2, extras):
    ne = len(extras)

    def body(*refs):
        (x_ref, s0_ref, s1_ref, cnt_ref, W2_ref, b2_ref, M1a_ref, M1b_ref,
         bm1_ref, g_ref, bln_ref, M2_ref, bm2_ref) = refs[:13]
        prefs = refs[13:13 + 2 * ne]
        outs = refs[13 + 2 * ne:]
        cntv = cnt_ref[...]
        seg_ = jnp.concatenate([s0_ref[...], s1_ref[...]],
                               axis=1).astype(jnp.float32)
        msg = (jnp.dot(seg_ / jnp.maximum(cntv, 1.0), W2_ref[...],
                       preferred_element_type=jnp.float32)
               + b2_ref[...] * (cntv > 0))
        t = (jnp.dot(x_ref[...], M1a_ref[...],
                     preferred_element_type=jnp.float32)
             + jnp.dot(msg, M1b_ref[...], preferred_element_type=jnp.float32)
             + bm1_ref[...])
        mu = jnp.mean(t, axis=-1, keepdims=True)
        var = jnp.mean((t - mu) ** 2, axis=-1, keepdims=True)
        h = jnp.maximum(
            g_ref[...] * (t - mu) / jnp.sqrt(var + 1e-5) + bln_ref[...], 0.0)
        y = jnp.dot(h, M2_ref[...],
                    preferred_element_type=jnp.float32) + bm2_ref[...]
        outs[0][...] = y
        for k in range(ne):
            outs[1 + k][...] = jnp.dot(
                y, prefs[2 * k][...],
                preferred_element_type=jnp.float32) + prefs[2 * k + 1][...]

    in_specs = [_rows(H), _rows(HH), _rows(HH), pl.BlockSpec((_RB, 1), lambda i: (i, 0)),
                _full((H, H)), _full((1, H)), _full((H, H)), _full((H, H)),
                _full((1, H)), _full((1, H)), _full((1, H)), _full((H, H)),
                _full((1, H))]
    p_ = jnp.arange(H)
    W2 = W2[(p_ // 32) * 32 + (p_ % 2) * 16 + (p_ % 32) // 2]
    args = [x, seg[0], seg[1], cnt, W2, b2.reshape(1, H), M1[:H], M1[H:],
            bm1.reshape(1, H), g.reshape(1, H), bln.reshape(1, H), M2,
            bm2.reshape(1, H)]
    for (P, p) in extras:
        in_specs += [_full((H, H)), _full((1, H))]
        args += [P, p.reshape(1, H)]
    out_shapes = tuple(jax.ShapeDtypeStruct((NN, H), jnp.float32)
                       for _ in range(1 + ne))
    out_specs = tuple(_rows(H) for _ in range(1 + ne))
    return pl.pallas_call(
        body, grid=(_NG,), in_specs=in_specs, out_specs=out_specs,
        out_shape=out_shapes)(*args)


def _pack_half(t32):
    lo = lax.bitcast_convert_type(t32[:, :16].astype(jnp.bfloat16),
                                  jnp.uint16).astype(jnp.uint32)
    hi = lax.bitcast_convert_type(t32[:, 16:].astype(jnp.bfloat16),
                                  jnp.uint16).astype(jnp.uint32)
    return (lo | (hi << 16)).astype(jnp.int32)


def _pack_table(t):
    return jnp.concatenate([_pack_half(t[:, :HH]), _pack_half(t[:, HH:])],
                           axis=0)


def _edge_phase(tableD, tableS, wa, dstg, srcg, agg3, ea3):
    wa_p = jnp.concatenate([_pack_half(wa[None, :HH]),
                            _pack_half(wa[None, HH:])], axis=0).reshape(-1)
    seg2 = _make_edge_seg_kernel()(_pack_table(tableD), _pack_table(tableS),
                                   dstg, srcg, agg3, ea3, wa_p)
    return seg2


def kernel(constraint_features, variable_features, edge_index, edge_attr,
           params):
    src = edge_index[0].astype(jnp.int32)
    dst = edge_index[1].astype(jnp.int32)
    ea = edge_attr.astype(jnp.float32)

    npad = EPAD - EE
    zpad = jnp.zeros((npad,), jnp.int32)
    dstp = jnp.concatenate([dst, zpad])
    srcp = jnp.concatenate([src, zpad])
    dsta = jnp.concatenate([dst, jnp.full((npad,), NN, jnp.int32)])
    srca = jnp.concatenate([src, jnp.full((npad,), NN, jnp.int32)])
    eap = jnp.concatenate([ea, jnp.zeros((npad,), jnp.float32)])
    dstg = jnp.concatenate([dstp, dstp + NN])
    srcg = jnp.concatenate([srcp, srcp + NN])
    dst3 = dsta.reshape(NR3, SUP, KC)
    src3 = srca.reshape(NR3, SUP, KC)
    ea3 = eap.reshape(NR3, SUP, KC)

    cnts = _make_count_kernel()(
        jnp.concatenate([dst3, src3]))
    cnt_c = cnts[0, :, 0:1]
    cnt_v = cnts[1, :, 0:1]

    lp0, lp1 = params['layers']
    W1_0 = lp0['cmsg1']['W']
    V1_0 = lp0['vmsg1']['W']
    W1_1 = lp1['cmsg1']['W']
    V1_1 = lp1['vmsg1']['W']

    cf = jnp.pad(constraint_features, ((0, 0), (0, 3)))
    vf = jnp.pad(variable_features, ((0, 0), (0, 5)))
    Wc = jnp.pad(params['cin']['W'], ((0, 3), (0, 0)))
    Wv = jnp.pad(params['vin']['W'], ((0, 5), (0, 0)))
    ch, A1 = _project(cf, Wc, params['cin']['b'],
                      [(W1_0[:H], jnp.zeros((H,), jnp.float32))])
    vh, B1, B2 = _project(
        vf, Wv, params['vin']['b'],
        [(W1_0[H:2 * H], lp0['cmsg1']['b']),
         (V1_0[:H], lp0['vmsg1']['b'])])

    seg = _edge_phase(A1, B1, W1_0[2 * H], dstg, srcg, dst3, ea3)
    ch, A2, A1n = _node_update(
        ch, seg, cnt_c, lp0['cmsg2']['W'], lp0['cmsg2']['b'],
        lp0['cmlp1']['W'], lp0['cmlp1']['b'], lp0['cln_g'], lp0['cln_b'],
        lp0['cmlp2']['W'], lp0['cmlp2']['b'],
        [(V1_0[H:2 * H], jnp.zeros((H,), jnp.float32)),
         (W1_1[:H], jnp.zeros((H,), jnp.float32))])

    seg = _edge_phase(A2, B2, V1_0[2 * H], dstg, srcg, src3, ea3)
    vh, B1n, B2n = _node_update(
        vh, seg, cnt_v, lp0['vmsg2']['W'], lp0['vmsg2']['b'],
        lp0['vmlp1']['W'], lp0['vmlp1']['b'], lp0['vln_g'], lp0['vln_b'],
        lp0['vmlp2']['W'], lp0['vmlp2']['b'],
        [(W1_1[H:2 * H], lp1['cmsg1']['b']),
         (V1_1[:H], lp1['vmsg1']['b'])])

    seg = _edge_phase(A1n, B1n, W1_1[2 * H], dstg, srcg, dst3, ea3)
    ch, A2n, out_c = _node_update(
        ch, seg, cnt_c, lp1['cmsg2']['W'], lp1['cmsg2']['b'],
        lp1['cmlp1']['W'], lp1['cmlp1']['b'], lp1['cln_g'], lp1['cln_b'],
        lp1['cmlp2']['W'], lp1['cmlp2']['b'],
        [(V1_1[H:2 * H], jnp.zeros((H,), jnp.float32)),
         (params['cout']['W'], params['cout']['b'])])

    seg = _edge_phase(A2n, B2n, V1_1[2 * H], dstg, srcg, src3, ea3)
    _, out_v = _node_update(
        vh, seg, cnt_v, lp1['vmsg2']['W'], lp1['vmsg2']['b'],
        lp1['vmlp1']['W'], lp1['vmlp1']['b'], lp1['vln_g'], lp1['vln_b'],
        lp1['vmlp2']['W'], lp1['vmlp2']['b'],
        [(params['vout']['W'], params['vout']['b'])])

    return out_c, out_v


# speedup vs baseline: 1.9935x; 1.2316x over previous
import functools

import jax
import jax.numpy as jnp
from jax import lax
from jax.experimental import pallas as pl
from jax.experimental.pallas import tpu as pltpu
from jax.experimental.pallas import tpu_sc as plsc

NN = 50000
EE = 800000
H = 64
HH = 32
NSUB = 16
KC = 128
SUP = 50
NSUP = 8
EPAD = NSUB * NSUP * SUP * KC
EPTP = EPAD // NSUB
NR3 = EPAD // (SUP * KC)
NACC = NN + 8
UR = 400
NUNITS = NN // UR

@functools.lru_cache(maxsize=None)
def _make_edge_seg_kernel():
    mesh = plsc.VectorSubcoreMesh(core_axis_name="c", subcore_axis_name="s")
    return functools.partial(
        pl.kernel,
        out_type=jax.ShapeDtypeStruct((2, NN, HH), jnp.bfloat16),
        mesh=mesh,
        scratch_types=[
            pltpu.VMEM((SUP * KC,), jnp.int32),
            pltpu.VMEM((SUP * KC,), jnp.int32),
            pltpu.VMEM((SUP, KC), jnp.int32),
            pltpu.VMEM((SUP, KC), jnp.float32),
            pltpu.VMEM((16,), jnp.int32),
            pltpu.VMEM((KC, 16), jnp.int32),
            pltpu.VMEM((KC, 16), jnp.int32),
            pltpu.VMEM((KC, 16), jnp.int32),
            pltpu.VMEM((KC, 16), jnp.int32),
            pltpu.VMEM((KC, HH), jnp.bfloat16),
            pltpu.VMEM((KC, HH), jnp.bfloat16),
            pltpu.VMEM((UR, HH), jnp.bfloat16),
            pltpu.VMEM_SHARED((NACC, HH), jnp.bfloat16),
            pltpu.SemaphoreType.DMA,
            pltpu.SemaphoreType.DMA,
            pltpu.SemaphoreType.DMA,
            pltpu.SemaphoreType.DMA,
            pltpu.SemaphoreType.DMA,
            pltpu.SemaphoreType.DMA,
        ],
        compiler_params=pltpu.CompilerParams(needs_layout_passes=False, use_tc_tiling_on_sc=False),
    )(_edge_seg_body)


def _edge_seg_body(tD, tS, dstg, srcg, agg3, ea3, wa, out,
                   dbuf, sbuf, aggbuf, eabuf, wabuf, Ab0, Ab1, Bb0, Bb1,
                   Mb0, Mb1, Zb, acc, sga0, sga1, sgb0, sgb1, ssc0, ssc1):
    c = lax.axis_index("c")
    s = lax.axis_index("s")
    zero16 = jnp.zeros((16,), jnp.float32)
    Abs_ = (Ab0, Ab1)
    Bbs = (Bb0, Bb1)
    Mbs = (Mb0, Mb1)
    sgas = (sga0, sga1)
    sgbs = (sgb0, sgb1)
    sscs = (ssc0, ssc1)

    with jax.named_scope("zero_acc"):
        zero32b = jnp.zeros((HH,), jnp.bfloat16)

        def zrow(i, carry):
            Zb[i, :] = zero32b
            return carry
        lax.fori_loop(0, UR, zrow, 0)

        nunits = (NUNITS - s + NSUB - 1) // NSUB

        def zcp(i, carry):
            pltpu.sync_copy(Zb, acc.at[pl.ds((s + NSUB * i) * UR, UR)])
            return carry
        lax.fori_loop(0, nunits, zcp, 0)

        pltpu.sync_copy(wa.at[pl.ds(c * 16, 16)], wabuf)
        wab = plsc.bitcast(wabuf[pl.ds(0, 16)], jnp.bfloat16)
        plsc.subcore_barrier()

    def fire_gathers(k, b):
        sl = pl.ds(k * KC, KC)
        pltpu.async_copy(tD.at[dbuf.at[sl]], Abs_[b], sgas[b])
        pltpu.async_copy(tS.at[sbuf.at[sl]], Bbs[b], sgbs[b])

    def wait_gathers(k, b):
        sl = pl.ds(k * KC, KC)
        pltpu.make_async_copy(tD.at[dbuf.at[sl]], Abs_[b], sgas[b]).wait()
        pltpu.make_async_copy(tS.at[sbuf.at[sl]], Bbs[b], sgbs[b]).wait()

    def superbody(sp, carry):
        off = c * EPAD + s * EPTP + sp * (SUP * KC)
        r = s * NSUP + sp
        pltpu.sync_copy(dstg.at[pl.ds(off, SUP * KC)], dbuf)
        pltpu.sync_copy(srcg.at[pl.ds(off, SUP * KC)], sbuf)
        pltpu.sync_copy(agg3.at[r], aggbuf)
        pltpu.sync_copy(ea3.at[r], eabuf)
        fire_gathers(0, 0)

        def pairbody(p, carry2):
            for b in range(2):
                k = 2 * p + b
                kn = jnp.minimum(k + 1, SUP - 1)

                @pl.when(k < SUP - 1)
                def _():
                    fire_gathers(kn, 1 - b)

                wait_gathers(k, b)

                @pl.when(k >= 2)
                def _():
                    pltpu.make_async_copy(Mbs[b], acc.at[aggbuf.at[k]],
                                          sscs[b]).wait()

                Ab = Abs_[b]
                Bb = Bbs[b]
                Mb = Mbs[b]
                kvec = jnp.full((16,), k, jnp.int32)

                def _ld(e):
                    eb = plsc.load_gather(
                        eabuf, [kvec, jnp.full((16,), e, jnp.int32)])
                    a32 = plsc.bitcast(Ab[e, pl.ds(0, 16)], jnp.bfloat16)
                    b32 = plsc.bitcast(Bb[e, pl.ds(0, 16)], jnp.bfloat16)
                    return eb, a32, b32

                eb, a32, b32 = _ld(0)
                for e in range(KC):
                    nxt = _ld(e + 1) if e < KC - 1 else None
                    ebb = plsc.pack(eb, eb, format=plsc.PackFormat.INTERLEAVED)
                    m32 = jnp.maximum(a32 + b32 + ebb * wab, 0.0)
                    Mb[e, :] = m32
                    if nxt is not None:
                        eb, a32, b32 = nxt
                pltpu.async_copy(Mb, acc.at[aggbuf.at[k]], sscs[b], add=True)
            return carry2
        lax.fori_loop(0, SUP // 2, pairbody, 0)

        for b in range(2):
            pltpu.make_async_copy(Mbs[b], acc.at[aggbuf.at[SUP - 2 + b]],
                                  sscs[b]).wait()
        return carry
    with jax.named_scope("edge_loop"):
        lax.fori_loop(0, NSUP, superbody, 0)
        plsc.subcore_barrier()

    with jax.named_scope("copy_out"):
        def ocp(i, carry):
            rs = (s + NSUB * i) * UR
            pltpu.sync_copy(acc.at[pl.ds(rs, UR)], out.at[c, pl.ds(rs, UR)])
            return carry
        lax.fori_loop(0, nunits, ocp, 0)


_CW = 16


@functools.lru_cache(maxsize=None)
def _make_count_kernel():
    mesh = plsc.VectorSubcoreMesh(core_axis_name="c", subcore_axis_name="s")
    return functools.partial(
        pl.kernel,
        out_type=jax.ShapeDtypeStruct((2, NN, _CW), jnp.float32),
        mesh=mesh,
        scratch_types=[
            pltpu.VMEM((SUP, KC), jnp.int32),
            pltpu.VMEM((KC, _CW), jnp.float32),
            pltpu.VMEM((UR, _CW), jnp.float32),
            pltpu.VMEM_SHARED((NACC, _CW), jnp.float32),
        ],
        compiler_params=pltpu.CompilerParams(needs_layout_passes=False, use_tc_tiling_on_sc=False),
    )(_count_body)


def _count_body(agg4, out, aggbuf, Ob, Zb, acc):
    c = lax.axis_index("c")
    s = lax.axis_index("s")
    zero16 = jnp.zeros((16,), jnp.float32)
    one16 = jnp.ones((16,), jnp.float32)

    def fill(i, carry):
        Zb[i, pl.ds(0, 16)] = zero16
        return carry
    lax.fori_loop(0, UR, fill, 0)

    def fillo(i, carry):
        Ob[i, pl.ds(0, 16)] = one16
        return carry
    lax.fori_loop(0, KC, fillo, 0)

    nunits = (NUNITS - s + NSUB - 1) // NSUB

    def zcp(i, carry):
        pltpu.sync_copy(Zb, acc.at[pl.ds((s + NSUB * i) * UR, UR)])
        return carry
    lax.fori_loop(0, nunits, zcp, 0)
    plsc.subcore_barrier()

    def superbody(sp, carry):
        r = c * NR3 + s * NSUP + sp
        pltpu.sync_copy(agg4.at[r], aggbuf)

        def chunkbody(k, carry2):
            pltpu.sync_copy(Ob, acc.at[aggbuf.at[k]], add=True)
            return carry2
        lax.fori_loop(0, SUP, chunkbody, 0)
        return carry
    with jax.named_scope("edge_loop"):
        lax.fori_loop(0, NSUP, superbody, 0)
        plsc.subcore_barrier()

    with jax.named_scope("copy_out"):
        def ocp(i, carry):
            rs = (s + NSUB * i) * UR
            pltpu.sync_copy(acc.at[pl.ds(rs, UR)], out.at[c, pl.ds(rs, UR)])
            return carry
        lax.fori_loop(0, nunits, ocp, 0)


_RB = 1000
_NG = NN // _RB


def _full(shape):
    return pl.BlockSpec(shape, lambda i: (0,) * len(shape))


def _rows(width):
    return pl.BlockSpec((_RB, width), lambda i: (i, 0))


def _project(x, W, b, extras):
    din = x.shape[1]
    ne = len(extras)

    def body(*refs):
        x_ref, W_ref, b_ref = refs[:3]
        prefs = refs[3:3 + 2 * ne]
        outs = refs[3 + 2 * ne:]
        y = jnp.dot(x_ref[...], W_ref[...],
                    preferred_element_type=jnp.float32) + b_ref[...]
        outs[0][...] = y
        for t in range(ne):
            outs[1 + t][...] = jnp.dot(
                y, prefs[2 * t][...],
                preferred_element_type=jnp.float32) + prefs[2 * t + 1][...]

    in_specs = [_rows(din), _full((din, H)), _full((1, H))]
    args = [x, W, b.reshape(1, H)]
    for (P, p) in extras:
        in_specs += [_full((H, H)), _full((1, H))]
        args += [P, p.reshape(1, H)]
    out_shapes = tuple(jax.ShapeDtypeStruct((NN, H), jnp.float32)
                       for _ in range(1 + ne))
    out_specs = tuple(_rows(H) for _ in range(1 + ne))
    return pl.pallas_call(
        body, grid=(_NG,), in_specs=in_specs, out_specs=out_specs,
        out_shape=out_shapes)(*args)


def _node_update(x, seg, cnt, W2, b2, M1, bm1, g, bln, M2, bm2, extras):
    ne = len(extras)

    def body(*refs):
        (x_ref, s0_ref, s1_ref, cnt_ref, W2_ref, b2_ref, M1a_ref, M1b_ref,
         bm1_ref, g_ref, bln_ref, M2_ref, bm2_ref) = refs[:13]
        prefs = refs[13:13 + 2 * ne]
        outs = refs[13 + 2 * ne:]
        cntv = cnt_ref[...]
        seg_ = jnp.concatenate([s0_ref[...], s1_ref[...]],
                               axis=1).astype(jnp.float32)
        msg = (jnp.dot(seg_ / jnp.maximum(cntv, 1.0), W2_ref[...],
                       preferred_element_type=jnp.float32)
               + b2_ref[...] * (cntv > 0))
        t = (jnp.dot(x_ref[...], M1a_ref[...],
                     preferred_element_type=jnp.float32)
             + jnp.dot(msg, M1b_ref[...], preferred_element_type=jnp.float32)
             + bm1_ref[...])
        mu = jnp.mean(t, axis=-1, keepdims=True)
        var = jnp.mean((t - mu) ** 2, axis=-1, keepdims=True)
        h = jnp.maximum(
            g_ref[...] * (t - mu) / jnp.sqrt(var + 1e-5) + bln_ref[...], 0.0)
        y = jnp.dot(h, M2_ref[...],
                    preferred_element_type=jnp.float32) + bm2_ref[...]
        outs[0][...] = y
        for k in range(ne):
            outs[1 + k][...] = jnp.dot(
                y, prefs[2 * k][...],
                preferred_element_type=jnp.float32) + prefs[2 * k + 1][...]

    in_specs = [_rows(H), _rows(HH), _rows(HH), pl.BlockSpec((_RB, 1), lambda i: (i, 0)),
                _full((H, H)), _full((1, H)), _full((H, H)), _full((H, H)),
                _full((1, H)), _full((1, H)), _full((1, H)), _full((H, H)),
                _full((1, H))]
    p_ = jnp.arange(H)
    W2 = W2[(p_ // 32) * 32 + (p_ % 2) * 16 + (p_ % 32) // 2]
    args = [x, seg[0], seg[1], cnt, W2, b2.reshape(1, H), M1[:H], M1[H:],
            bm1.reshape(1, H), g.reshape(1, H), bln.reshape(1, H), M2,
            bm2.reshape(1, H)]
    for (P, p) in extras:
        in_specs += [_full((H, H)), _full((1, H))]
        args += [P, p.reshape(1, H)]
    out_shapes = tuple(jax.ShapeDtypeStruct((NN, H), jnp.float32)
                       for _ in range(1 + ne))
    out_specs = tuple(_rows(H) for _ in range(1 + ne))
    return pl.pallas_call(
        body, grid=(_NG,), in_specs=in_specs, out_specs=out_specs,
        out_shape=out_shapes)(*args)


def _pack_half(t32):
    lo = lax.bitcast_convert_type(t32[:, :16].astype(jnp.bfloat16),
                                  jnp.uint16).astype(jnp.uint32)
    hi = lax.bitcast_convert_type(t32[:, 16:].astype(jnp.bfloat16),
                                  jnp.uint16).astype(jnp.uint32)
    return (lo | (hi << 16)).astype(jnp.int32)


def _pack_table(t):
    return jnp.concatenate([_pack_half(t[:, :HH]), _pack_half(t[:, HH:])],
                           axis=0)


def _edge_phase(tableD, tableS, wa, dstg, srcg, agg3, ea3):
    wa_p = jnp.concatenate([_pack_half(wa[None, :HH]),
                            _pack_half(wa[None, HH:])], axis=0).reshape(-1)
    seg2 = _make_edge_seg_kernel()(_pack_table(tableD), _pack_table(tableS),
                                   dstg, srcg, agg3, ea3, wa_p)
    return seg2


def kernel(constraint_features, variable_features, edge_index, edge_attr,
           params):
    src = edge_index[0].astype(jnp.int32)
    dst = edge_index[1].astype(jnp.int32)
    ea = edge_attr.astype(jnp.float32)

    npad = EPAD - EE
    zpad = jnp.zeros((npad,), jnp.int32)
    dstp = jnp.concatenate([dst, zpad])
    srcp = jnp.concatenate([src, zpad])
    dsta = jnp.concatenate([dst, jnp.full((npad,), NN, jnp.int32)])
    srca = jnp.concatenate([src, jnp.full((npad,), NN, jnp.int32)])
    eap = jnp.concatenate([ea, jnp.zeros((npad,), jnp.float32)])
    dstg = jnp.concatenate([dstp, dstp + NN])
    srcg = jnp.concatenate([srcp, srcp + NN])
    dst3 = dsta.reshape(NR3, SUP, KC)
    src3 = srca.reshape(NR3, SUP, KC)
    ea3 = eap.reshape(NR3, SUP, KC)

    cnts = _make_count_kernel()(
        jnp.concatenate([dst3, src3]))
    cnt_c = cnts[0, :, 0:1]
    cnt_v = cnts[1, :, 0:1]

    lp0, lp1 = params['layers']
    W1_0 = lp0['cmsg1']['W']
    V1_0 = lp0['vmsg1']['W']
    W1_1 = lp1['cmsg1']['W']
    V1_1 = lp1['vmsg1']['W']

    cf = jnp.pad(constraint_features, ((0, 0), (0, 3)))
    vf = jnp.pad(variable_features, ((0, 0), (0, 5)))
    Wc = jnp.pad(params['cin']['W'], ((0, 3), (0, 0)))
    Wv = jnp.pad(params['vin']['W'], ((0, 5), (0, 0)))
    ch, A1 = _project(cf, Wc, params['cin']['b'],
                      [(W1_0[:H], jnp.zeros((H,), jnp.float32))])
    vh, B1, B2 = _project(
        vf, Wv, params['vin']['b'],
        [(W1_0[H:2 * H], lp0['cmsg1']['b']),
         (V1_0[:H], lp0['vmsg1']['b'])])

    seg = _edge_phase(A1, B1, W1_0[2 * H], dstg, srcg, dst3, ea3)
    ch, A2, A1n = _node_update(
        ch, seg, cnt_c, lp0['cmsg2']['W'], lp0['cmsg2']['b'],
        lp0['cmlp1']['W'], lp0['cmlp1']['b'], lp0['cln_g'], lp0['cln_b'],
        lp0['cmlp2']['W'], lp0['cmlp2']['b'],
        [(V1_0[H:2 * H], jnp.zeros((H,), jnp.float32)),
         (W1_1[:H], jnp.zeros((H,), jnp.float32))])

    seg = _edge_phase(A2, B2, V1_0[2 * H], dstg, srcg, src3, ea3)
    vh, B1n, B2n = _node_update(
        vh, seg, cnt_v, lp0['vmsg2']['W'], lp0['vmsg2']['b'],
        lp0['vmlp1']['W'], lp0['vmlp1']['b'], lp0['vln_g'], lp0['vln_b'],
        lp0['vmlp2']['W'], lp0['vmlp2']['b'],
        [(W1_1[H:2 * H], lp1['cmsg1']['b']),
         (V1_1[:H], lp1['vmsg1']['b'])])

    seg = _edge_phase(A1n, B1n, W1_1[2 * H], dstg, srcg, dst3, ea3)
    ch, A2n, out_c = _node_update(
        ch, seg, cnt_c, lp1['cmsg2']['W'], lp1['cmsg2']['b'],
        lp1['cmlp1']['W'], lp1['cmlp1']['b'], lp1['cln_g'], lp1['cln_b'],
        lp1['cmlp2']['W'], lp1['cmlp2']['b'],
        [(V1_1[H:2 * H], jnp.zeros((H,), jnp.float32)),
         (params['cout']['W'], params['cout']['b'])])

    seg = _edge_phase(A2n, B2n, V1_1[2 * H], dstg, srcg, src3, ea3)
    _, out_v = _node_update(
        vh, seg, cnt_v, lp1['vmsg2']['W'], lp1['vmsg2']['b'],
        lp1['vmlp1']['W'], lp1['vmlp1']['b'], lp1['vln_g'], lp1['vln_b'],
        lp1['vmlp2']['W'], lp1['vmlp2']['b'],
        [(params['vout']['W'], params['vout']['b'])])

    return out_c, out_v
